# baseline (speedup 1.0000x reference)
import dataclasses
import functools

import jax
import jax.numpy as jnp
from jax import lax
from jax.experimental import pallas as pl
from jax.experimental.pallas import tpu as pltpu
from jax.experimental.pallas import tpu_sc as plsc

_N = 100000
_K = 64
_ROWS = 400
_CHUNKS = _N // _ROWS
_NW = 32
_LANES = 16
_UNROLL = 4


def _sc_body(d_hbm, r_hbm, s_hbm, o_hbm, dist_v, rho_v, sig_v, out_v):
    wid = lax.axis_index("s") * 2 + lax.axis_index("c")
    n_own = jnp.where(wid < _CHUNKS % _NW, _CHUNKS // _NW + 1, _CHUNKS // _NW)

    def chunk_step(i, _):
        g = wid + i * _NW
        row0 = g * _ROWS
        el0 = row0 * _K
        pltpu.sync_copy(d_hbm.at[pl.ds(el0, _ROWS * _K)], dist_v)
        pltpu.sync_copy(r_hbm.at[pl.ds(row0, _ROWS)], rho_v)
        pltpu.sync_copy(s_hbm.at[pl.ds(row0, _ROWS)], sig_v)

        def row_step(r0, _):
            for u in range(_UNROLL):
                r = r0 * _UNROLL + u
                idx = jnp.full((_LANES,), r, jnp.int32)
                rho_b = plsc.load_gather(rho_v, [idx])
                sig_b = plsc.load_gather(sig_v, [idx])
                rinv_b = jnp.float32(1.0) / sig_b
                for c in range(_K // _LANES):
                    sl = pl.ds(r * _K + c * _LANES, _LANES)
                    x = dist_v[sl]
                    out_v[sl] = jnp.minimum(jnp.exp((rho_b - x) * rinv_b), 1.0)
            return 0

        lax.fori_loop(0, _ROWS // _UNROLL, row_step, 0)

        pltpu.sync_copy(out_v, o_hbm.at[pl.ds(el0, _ROWS * _K)])
        return 0

    lax.fori_loop(0, n_own, chunk_step, 0)


def kernel(distances, rho, sigma):
    n, k = distances.shape
    mesh = plsc.VectorSubcoreMesh(core_axis_name="c", subcore_axis_name="s")
    cp = pltpu.CompilerParams()
    if "needs_layout_passes" in pltpu.CompilerParams.__dataclass_fields__:
        cp = dataclasses.replace(cp, needs_layout_passes=False)
    run = functools.partial(
        pl.kernel,
        out_type=jax.ShapeDtypeStruct((n * k,), jnp.float32),
        mesh=mesh,
        scratch_types=[
            pltpu.VMEM((_ROWS * _K,), jnp.float32),
            pltpu.VMEM((_ROWS,), jnp.float32),
            pltpu.VMEM((_ROWS,), jnp.float32),
            pltpu.VMEM((_ROWS * _K,), jnp.float32),
        ],
        compiler_params=cp,
    )(_sc_body)
    out = run(distances.reshape(-1), rho, sigma)
    return out.reshape(n, k)


# speedup vs baseline: 1.0925x; 1.0925x over previous
import dataclasses
import functools

import jax
import jax.numpy as jnp
from jax import lax
from jax.experimental import pallas as pl
from jax.experimental.pallas import tpu as pltpu
from jax.experimental.pallas import tpu_sc as plsc

_N = 100000
_K = 64
_ROWS = 400
_CHUNKS = _N // _ROWS
_NW = 32
_LANES = 16
_UNROLL = 8
_SLOTS = (_CHUNKS + _NW - 1) // _NW


def _sc_body(d_hbm, r_hbm, s_hbm, o_hbm,
             dist0, out0, rho0, sig0, dist1, out1, rho1, sig1,
             si0, si1, so0, so1):
    wid = lax.axis_index("s") * 2 + lax.axis_index("c")
    bufs = ((dist0, out0, rho0, sig0, si0, so0),
            (dist1, out1, rho1, sig1, si1, so1))

    def start_in(g, b):
        dist_v, _, rho_v, sig_v, si, _ = bufs[b]
        pltpu.async_copy(d_hbm.at[pl.ds(g * _ROWS * _K, _ROWS * _K)], dist_v, si)
        pltpu.async_copy(r_hbm.at[pl.ds(g * _ROWS, _ROWS)], rho_v, si)
        pltpu.async_copy(s_hbm.at[pl.ds(g * _ROWS, _ROWS)], sig_v, si)

    def wait_in(b):
        dist_v, _, rho_v, sig_v, si, _ = bufs[b]
        pltpu.make_async_copy(d_hbm.at[pl.ds(0, _ROWS * _K)], dist_v, si).wait()
        pltpu.make_async_copy(r_hbm.at[pl.ds(0, _ROWS)], rho_v, si).wait()
        pltpu.make_async_copy(s_hbm.at[pl.ds(0, _ROWS)], sig_v, si).wait()

    def start_out(g, b):
        _, out_v, _, _, _, so = bufs[b]
        pltpu.async_copy(out_v, o_hbm.at[pl.ds(g * _ROWS * _K, _ROWS * _K)], so)

    def wait_out(b):
        _, out_v, _, _, _, so = bufs[b]
        pltpu.make_async_copy(out_v, o_hbm.at[pl.ds(0, _ROWS * _K)], so).wait()

    def compute(b):
        dist_v, out_v, rho_v, sig_v, _, _ = bufs[b]

        def row_step(r0, _):
            for u in range(_UNROLL):
                r = r0 * _UNROLL + u
                idx = jnp.full((_LANES,), r, jnp.int32)
                rho_b = plsc.load_gather(rho_v, [idx])
                sig_b = plsc.load_gather(sig_v, [idx])
                rinv_b = jnp.float32(1.0) / sig_b
                for c in range(_K // _LANES):
                    sl = pl.ds(r * _K + c * _LANES, _LANES)
                    x = dist_v[sl]
                    out_v[sl] = jnp.minimum(jnp.exp((rho_b - x) * rinv_b), 1.0)
            return 0

        lax.fori_loop(0, _ROWS // _UNROLL, row_step, 0)

    start_in(wid, 0)
    start_in(wid + _NW, 1)

    def pair_step(t, _):
        for par in (0, 1):
            g = wid + (2 * t + par) * _NW

            @pl.when(g < _CHUNKS)
            def _():
                @pl.when(t >= 1)
                def _():
                    wait_out(par)

                wait_in(par)
                compute(par)
                start_out(g, par)

                @pl.when(g + 2 * _NW < _CHUNKS)
                def _():
                    start_in(g + 2 * _NW, par)
        return 0

    lax.fori_loop(0, (_SLOTS + 1) // 2, pair_step, 0)
    wait_out(0)
    wait_out(1)


def kernel(distances, rho, sigma):
    n, k = distances.shape
    mesh = plsc.VectorSubcoreMesh(core_axis_name="c", subcore_axis_name="s")
    cp = pltpu.CompilerParams()
    if "needs_layout_passes" in pltpu.CompilerParams.__dataclass_fields__:
        cp = dataclasses.replace(cp, needs_layout_passes=False)
    run = functools.partial(
        pl.kernel,
        out_type=jax.ShapeDtypeStruct((n * k,), jnp.float32),
        mesh=mesh,
        scratch_types=[
            pltpu.VMEM((_ROWS * _K,), jnp.float32),
            pltpu.VMEM((_ROWS * _K,), jnp.float32),
            pltpu.VMEM((_ROWS,), jnp.float32),
            pltpu.VMEM((_ROWS,), jnp.float32),
            pltpu.VMEM((_ROWS * _K,), jnp.float32),
            pltpu.VMEM((_ROWS * _K,), jnp.float32),
            pltpu.VMEM((_ROWS,), jnp.float32),
            pltpu.VMEM((_ROWS,), jnp.float32),
            pltpu.SemaphoreType.DMA,
            pltpu.SemaphoreType.DMA,
            pltpu.SemaphoreType.DMA,
            pltpu.SemaphoreType.DMA,
        ],
        compiler_params=cp,
    )(_sc_body)
    out = run(distances.reshape(-1), rho, sigma)
    return out.reshape(n, k)


# speedup vs baseline: 1.4157x; 1.2959x over previous
import dataclasses
import functools

import jax
import jax.numpy as jnp
from jax import lax
from jax.experimental import pallas as pl
from jax.experimental.pallas import tpu as pltpu
from jax.experimental.pallas import tpu_sc as plsc

_N = 100000
_K = 64
_ROWS = 400
_CHUNKS = _N // _ROWS
_NW = 32
_LANES = 16
_UNROLL = 2
_SLOTS = (_CHUNKS + _NW - 1) // _NW

_LOG2E = jnp.float32(1.4426950408889634)
_C1 = jnp.float32(0.6931471805599453)
_C2 = jnp.float32(0.2402265069591007)
_C3 = jnp.float32(0.05550410866482158)
_C4 = jnp.float32(0.009618129107628477)


def _sc_body(d_hbm, r_hbm, s_hbm, o_hbm,
             dist0, out0, rho0, sig0, dist1, out1, rho1, sig1,
             si0, si1, so0, so1):
    wid = lax.axis_index("s") * 2 + lax.axis_index("c")
    bufs = ((dist0, out0, rho0, sig0, si0, so0),
            (dist1, out1, rho1, sig1, si1, so1))

    def start_in(g, b):
        dist_v, _, rho_v, sig_v, si, _ = bufs[b]
        pltpu.async_copy(d_hbm.at[pl.ds(g * _ROWS * _K, _ROWS * _K)], dist_v, si)
        pltpu.async_copy(r_hbm.at[pl.ds(g * _ROWS, _ROWS)], rho_v, si)
        pltpu.async_copy(s_hbm.at[pl.ds(g * _ROWS, _ROWS)], sig_v, si)

    def wait_in(b):
        dist_v, _, rho_v, sig_v, si, _ = bufs[b]
        pltpu.make_async_copy(d_hbm.at[pl.ds(0, _ROWS * _K)], dist_v, si).wait()
        pltpu.make_async_copy(r_hbm.at[pl.ds(0, _ROWS)], rho_v, si).wait()
        pltpu.make_async_copy(s_hbm.at[pl.ds(0, _ROWS)], sig_v, si).wait()

    def start_out(g, b):
        _, out_v, _, _, _, so = bufs[b]
        pltpu.async_copy(out_v, o_hbm.at[pl.ds(g * _ROWS * _K, _ROWS * _K)], so)

    def wait_out(b):
        _, out_v, _, _, _, so = bufs[b]
        pltpu.make_async_copy(out_v, o_hbm.at[pl.ds(0, _ROWS * _K)], so).wait()

    def compute(b):
        dist_v, out_v, rho_v, sig_v, _, _ = bufs[b]

        def row_step(r0, _):
            for u in range(_UNROLL):
                r = r0 * _UNROLL + u
                idx = jnp.full((_LANES,), r, jnp.int32)
                rho_b = plsc.load_gather(rho_v, [idx])
                sig_b = plsc.load_gather(sig_v, [idx])
                rinv2 = _LOG2E / sig_b
                nc = _K // _LANES
                sls = [pl.ds(r * _K + c * _LANES, _LANES) for c in range(nc)]
                xs = [dist_v[sl] for sl in sls]
                ts = [jnp.minimum((rho_b - x) * rinv2, jnp.float32(0.0)) for x in xs]
                ts = [jnp.maximum(t, jnp.float32(-126.0)) for t in ts]
                ns = [(t - jnp.float32(0.5)).astype(jnp.int32) for t in ts]
                fs = [t - n.astype(jnp.float32) for t, n in zip(ts, ns)]
                ps = [_C4 * f + _C3 for f in fs]
                ps = [p * f + _C2 for p, f in zip(ps, fs)]
                ps = [p * f + _C1 for p, f in zip(ps, fs)]
                ps = [p * f + jnp.float32(1.0) for p, f in zip(ps, fs)]
                scs = [plsc.bitcast((n + 127) << 23, jnp.float32) for n in ns]
                for sl, p, sc in zip(sls, ps, scs):
                    out_v[sl] = p * sc
            return 0

        lax.fori_loop(0, _ROWS // _UNROLL, row_step, 0)

    start_in(wid, 0)
    start_in(wid + _NW, 1)

    def pair_step(t, _):
        for par in (0, 1):
            g = wid + (2 * t + par) * _NW

            @pl.when(g < _CHUNKS)
            def _():
                @pl.when(t >= 1)
                def _():
                    wait_out(par)

                wait_in(par)
                compute(par)
                start_out(g, par)

                @pl.when(g + 2 * _NW < _CHUNKS)
                def _():
                    start_in(g + 2 * _NW, par)
        return 0

    lax.fori_loop(0, (_SLOTS + 1) // 2, pair_step, 0)
    wait_out(0)
    wait_out(1)


def kernel(distances, rho, sigma):
    n, k = distances.shape
    mesh = plsc.VectorSubcoreMesh(core_axis_name="c", subcore_axis_name="s")
    cp = pltpu.CompilerParams()
    if "needs_layout_passes" in pltpu.CompilerParams.__dataclass_fields__:
        cp = dataclasses.replace(cp, needs_layout_passes=False)
    run = functools.partial(
        pl.kernel,
        out_type=jax.ShapeDtypeStruct((n * k,), jnp.float32),
        mesh=mesh,
        scratch_types=[
            pltpu.VMEM((_ROWS * _K,), jnp.float32),
            pltpu.VMEM((_ROWS * _K,), jnp.float32),
            pltpu.VMEM((_ROWS,), jnp.float32),
            pltpu.VMEM((_ROWS,), jnp.float32),
            pltpu.VMEM((_ROWS * _K,), jnp.float32),
            pltpu.VMEM((_ROWS * _K,), jnp.float32),
            pltpu.VMEM((_ROWS,), jnp.float32),
            pltpu.VMEM((_ROWS,), jnp.float32),
            pltpu.SemaphoreType.DMA,
            pltpu.SemaphoreType.DMA,
            pltpu.SemaphoreType.DMA,
            pltpu.SemaphoreType.DMA,
        ],
        compiler_params=cp,
    )(_sc_body)
    out = run(distances.reshape(-1), rho, sigma)
    return out.reshape(n, k)
